# baseline (speedup 1.0000x reference)
import functools

import jax
import jax.numpy as jnp
from jax import lax
from jax.experimental import pallas as pl
from jax.experimental.pallas import tpu as pltpu
from jax.experimental.pallas import tpu_sc as plsc

_NC = 2
_NS = 16
_L = 16
_NW = _NC * _NS


@functools.partial(jax.jit, static_argnums=(3, 4, 5))
def _gather_project(tableT, wb, xf, LEN, D, VP):
    B = xf.shape[0]
    per_w = B // _NW
    chunk = 64
    n_chunks = per_w // chunk
    n_full = LEN // _L
    tail = LEN - n_full * _L
    mesh = plsc.VectorSubcoreMesh(core_axis_name="c", subcore_axis_name="s")

    @functools.partial(
        pl.kernel,
        out_type=jax.ShapeDtypeStruct((B, LEN), jnp.float32),
        mesh=mesh,
        compiler_params=pltpu.CompilerParams(
            needs_layout_passes=False, use_tc_tiling_on_sc=True),
        scratch_types=[
            pltpu.VMEM((D, VP), jnp.float32),
            pltpu.VMEM((D + 1, 128), jnp.float32),
            pltpu.VMEM((VP,), jnp.float32),
            pltpu.VMEM((2, chunk, LEN), jnp.int32),
            pltpu.VMEM((2, chunk, LEN), jnp.float32),
            pltpu.SemaphoreType.DMA,
            pltpu.SemaphoreType.DMA,
            pltpu.SemaphoreType.DMA,
            pltpu.SemaphoreType.DMA,
        ],
    )
    def body(tableT_hbm, wb_hbm, x_hbm, out_hbm, tableT_v, wb_v, proj_v,
             idx_v, val_v, in_sem0, in_sem1, out_sem0, out_sem1):
        pltpu.sync_copy(tableT_hbm, tableT_v)
        pltpu.sync_copy(wb_hbm, wb_v)
        bias = wb_v[D, pl.ds(0, _L)]
        w_bcast = [wb_v[d, pl.ds(0, _L)] for d in range(D)]
        for g in range(VP // _L):
            acc = bias
            for d in range(D):
                acc = acc + tableT_v[d, pl.ds(g * _L, _L)] * w_bcast[d]
            proj_v[pl.ds(g * _L, _L)] = acc

        wid = lax.axis_index("s") * _NC + lax.axis_index("c")
        row0 = wid * per_w
        in_sems = [in_sem0, in_sem1]
        out_sems = [out_sem0, out_sem1]
        in_desc = [None, None]
        out_desc = [None, None]

        col_starts = [c * _L for c in range(n_full)]
        if tail:
            col_starts.append(LEN - _L)

        in_desc[0] = pltpu.async_copy(
            x_hbm.at[pl.ds(row0, chunk), :], idx_v.at[0], in_sems[0])

        for kk in range(n_chunks):
            buf = kk % 2
            nbuf = (kk + 1) % 2
            if kk + 1 < n_chunks:
                in_desc[nbuf] = pltpu.async_copy(
                    x_hbm.at[pl.ds(row0 + (kk + 1) * chunk, chunk), :],
                    idx_v.at[nbuf], in_sems[nbuf])
            in_desc[buf].wait()
            if out_desc[buf] is not None:
                out_desc[buf].wait()

            @plsc.parallel_loop(0, chunk, 1)
            def gather_row(r):
                for c0 in col_starts:
                    iv = idx_v[buf, r, pl.ds(c0, _L)]
                    val_v[buf, r, pl.ds(c0, _L)] = plsc.load_gather(
                        proj_v, [iv])

            out_desc[buf] = pltpu.async_copy(
                val_v.at[buf],
                out_hbm.at[pl.ds(row0 + kk * chunk, chunk), :], out_sems[buf])

        for buf in range(2):
            if out_desc[buf] is not None:
                out_desc[buf].wait()

    return body(tableT, wb, xf)


def kernel(x, embed_table, lin_w, lin_b):
    B, L = x.shape
    V, D = embed_table.shape
    VP = -(-V // 128) * 128
    xf = x.astype(jnp.int32)
    tableT = jnp.zeros((D, VP), jnp.float32).at[:, :V].set(
        embed_table.T.astype(jnp.float32))
    wvals = jnp.concatenate(
        [lin_w[0].astype(jnp.float32), lin_b.astype(jnp.float32)])
    wb = jnp.broadcast_to(wvals[:, None], (D + 1, 128))
    out = _gather_project(tableT, wb, xf, L, D, VP)
    return out.reshape(B, L, 1)


# speedup vs baseline: 1.0039x; 1.0039x over previous
import functools

import jax
import jax.numpy as jnp
from jax import lax
from jax.experimental import pallas as pl
from jax.experimental.pallas import tpu as pltpu
from jax.experimental.pallas import tpu_sc as plsc

_NC = 2
_NS = 16
_L = 16
_NW = _NC * _NS


@functools.partial(jax.jit, static_argnums=(3, 4, 5))
def _gather_project(tableT, wb, xf, LEN, D, VP):
    B = xf.shape[0]
    per_w = B // _NW
    chunk = 64
    n_chunks = per_w // chunk
    n_full = LEN // _L
    tail = LEN - n_full * _L
    mesh = plsc.VectorSubcoreMesh(core_axis_name="c", subcore_axis_name="s")

    @functools.partial(
        pl.kernel,
        out_type=jax.ShapeDtypeStruct((B, LEN), jnp.float32),
        mesh=mesh,
        compiler_params=pltpu.CompilerParams(needs_layout_passes=False),
        scratch_types=[
            pltpu.VMEM((D, VP), jnp.float32),
            pltpu.VMEM((D + 1, 128), jnp.float32),
            pltpu.VMEM((VP,), jnp.float32),
            pltpu.VMEM((2, chunk, LEN), jnp.int32),
            pltpu.VMEM((2, chunk, LEN), jnp.float32),
            pltpu.SemaphoreType.DMA,
            pltpu.SemaphoreType.DMA,
            pltpu.SemaphoreType.DMA,
            pltpu.SemaphoreType.DMA,
        ],
    )
    def body(tableT_hbm, wb_hbm, x_hbm, out_hbm, tableT_v, wb_v, proj_v,
             idx_v, val_v, in_sem0, in_sem1, out_sem0, out_sem1):
        wid = lax.axis_index("s") * _NC + lax.axis_index("c")
        row0 = wid * per_w
        in_sems = [in_sem0, in_sem1]
        out_sems = [out_sem0, out_sem1]
        in_desc = [None, None]
        out_desc = [None, None]

        in_desc[0] = pltpu.async_copy(
            x_hbm.at[pl.ds(row0, chunk), :], idx_v.at[0], in_sems[0])

        pltpu.sync_copy(tableT_hbm, tableT_v)
        pltpu.sync_copy(wb_hbm, wb_v)
        bias = wb_v[D, pl.ds(0, _L)]
        w_bcast = [wb_v[d, pl.ds(0, _L)] for d in range(D)]
        for g in range(VP // _L):
            acc = bias
            for d in range(D):
                acc = acc + tableT_v[d, pl.ds(g * _L, _L)] * w_bcast[d]
            proj_v[pl.ds(g * _L, _L)] = acc

        col_starts = [c * _L for c in range(n_full)]
        if tail:
            col_starts.append(LEN - _L)

        for kk in range(n_chunks):
            buf = kk % 2
            nbuf = (kk + 1) % 2
            if kk + 1 < n_chunks:
                in_desc[nbuf] = pltpu.async_copy(
                    x_hbm.at[pl.ds(row0 + (kk + 1) * chunk, chunk), :],
                    idx_v.at[nbuf], in_sems[nbuf])
            in_desc[buf].wait()
            if out_desc[buf] is not None:
                out_desc[buf].wait()

            @plsc.parallel_loop(0, chunk, 1, unroll=2)
            def gather_row(r):
                for c0 in col_starts:
                    iv = idx_v[buf, r, pl.ds(c0, _L)]
                    val_v[buf, r, pl.ds(c0, _L)] = plsc.load_gather(
                        proj_v, [iv])

            out_desc[buf] = pltpu.async_copy(
                val_v.at[buf],
                out_hbm.at[pl.ds(row0 + kk * chunk, chunk), :], out_sems[buf])

        for buf in range(2):
            if out_desc[buf] is not None:
                out_desc[buf].wait()

    return body(tableT, wb, xf)


def kernel(x, embed_table, lin_w, lin_b):
    B, L = x.shape
    V, D = embed_table.shape
    VP = -(-V // 128) * 128
    xf = x.astype(jnp.int32)
    tableT = jnp.zeros((D, VP), jnp.float32).at[:, :V].set(
        embed_table.T.astype(jnp.float32))
    wvals = jnp.concatenate(
        [lin_w[0].astype(jnp.float32), lin_b.astype(jnp.float32)])
    wb = jnp.broadcast_to(wvals[:, None], (D + 1, 128))
    out = _gather_project(tableT, wb, xf, L, D, VP)
    return out.reshape(B, L, 1)
